# Initial kernel scaffold; baseline (speedup 1.0000x reference)
import jax
import jax.numpy as jnp
from jax.experimental import pallas as pl


def kernel(x, batch_edge_index, batch_edge_types, ln1_g, ln1_b, W, root, bias, ln2_g, ln2_b, head_w, head_b):
    raise NotImplementedError("write your pallas kernel here")


# speedup vs baseline: 13.8562x; 13.8562x over previous
import functools

import jax
import jax.numpy as jnp
from jax import lax
from jax.experimental import pallas as pl
from jax.experimental.pallas import tpu as pltpu
from jax.experimental.pallas import tpu_sc as plsc

NC = 2
NS = 16
L = 16
NW = NC * NS


def _phase_a_body(x_ref, g_ref, b_ref, w_ref, root_ref, bias_ref,
                  z_ref, outroot_ref):
    xb = x_ref[...]
    m = jnp.mean(xb, axis=1, keepdims=True)
    v = jnp.mean((xb - m) * (xb - m), axis=1, keepdims=True)
    xf = (xb - m) * lax.rsqrt(v + 1e-5) * g_ref[...] + b_ref[...]
    outroot_ref[...] = (
        jnp.dot(xf, root_ref[...], preferred_element_type=jnp.float32,
                precision=lax.Precision.HIGHEST)
        + bias_ref[...])
    r = w_ref.shape[0]
    c = xb.shape[1]
    for i in range(r):
        z_ref[:, i * c:(i + 1) * c] = jnp.dot(
            xf, w_ref[i], preferred_element_type=jnp.float32,
            precision=lax.Precision.HIGHEST)


def _phase_a(xf2, g, b, w, root, bias):
    m, c = xf2.shape
    r = w.shape[0]
    bm = 1000
    grid = m // bm
    return pl.pallas_call(
        _phase_a_body,
        grid=(grid,),
        in_specs=[
            pl.BlockSpec((bm, c), lambda i: (i, 0)),
            pl.BlockSpec((1, c), lambda i: (0, 0)),
            pl.BlockSpec((1, c), lambda i: (0, 0)),
            pl.BlockSpec((r, c, c), lambda i: (0, 0, 0)),
            pl.BlockSpec((c, c), lambda i: (0, 0)),
            pl.BlockSpec((1, c), lambda i: (0, 0)),
        ],
        out_specs=[
            pl.BlockSpec((bm, r * c), lambda i: (i, 0)),
            pl.BlockSpec((bm, c), lambda i: (i, 0)),
        ],
        out_shape=[
            jax.ShapeDtypeStruct((m, r * c), jnp.float32),
            jax.ShapeDtypeStruct((m, c), jnp.float32),
        ],
    )(xf2, g.reshape(1, c), b.reshape(1, c), w, root, bias.reshape(1, c))


def _make_count_kernel(m_nodes, e_per_w, kc):
    zc = 200
    nchunk = m_nodes // zc
    iters = -(-nchunk // NS)
    mesh = plsc.VectorSubcoreMesh(core_axis_name="c", subcore_axis_name="s")

    @functools.partial(
        pl.kernel,
        out_type=jax.ShapeDtypeStruct((NC, m_nodes, 128), jnp.float32),
        mesh=mesh,
        scratch_types=[
            pltpu.VMEM((kc,), jnp.int32),
            pltpu.VMEM((kc,), jnp.int32),
            pltpu.VMEM((kc, 128), jnp.float32),
            pltpu.VMEM((zc, 128), jnp.float32),
            pltpu.VMEM_SHARED((m_nodes, 128), jnp.float32),
        ],
        compiler_params=pltpu.CompilerParams(needs_layout_passes=False),
    )
    def count_kernel(dst_hbm, typ_hbm, out_hbm, dstb, typb, stag, bounce,
                     cnt_sh):
        cidx = lax.axis_index("c")
        sidx = lax.axis_index("s")
        wid = sidx * NC + cidx

        for i in range(kc):
            for j in range(1, 8):
                stag[i, pl.ds(j * L, L)] = jnp.zeros((L,), jnp.float32)

        def zrow(i, carry):
            for j in range(8):
                bounce[i, pl.ds(j * L, L)] = jnp.zeros((L,), jnp.float32)
            return carry

        lax.fori_loop(0, zc, zrow, 0)
        for t in range(iters):
            ch = sidx + t * NS

            @pl.when(ch < nchunk)
            def _():
                pltpu.sync_copy(bounce, cnt_sh.at[pl.ds(ch * zc, zc)])
        plsc.subcore_barrier()

        base = wid * e_per_w

        def chunk(g, carry):
            off = base + g * kc
            pltpu.sync_copy(dst_hbm.at[pl.ds(off, kc)], dstb)
            pltpu.sync_copy(typ_hbm.at[pl.ds(off, kc)], typb)

            lanes = lax.iota(jnp.int32, L)
            for j in range(kc // L):
                tv = typb[pl.ds(j * L, L)]
                for e2 in range(L):
                    stag[j * L + e2, pl.ds(0, L)] = jnp.where(
                        lanes == tv[e2], 1.0, 0.0).astype(jnp.float32)
            pltpu.sync_copy(stag, cnt_sh.at[dstb], add=True)
            return carry

        lax.fori_loop(0, e_per_w // kc, chunk, 0)
        plsc.subcore_barrier()
        for t in range(iters):
            ch = sidx + t * NS

            @pl.when(ch < nchunk)
            def _():
                pltpu.sync_copy(cnt_sh.at[pl.ds(ch * zc, zc)], bounce)
                pltpu.sync_copy(bounce,
                                out_hbm.at[cidx, pl.ds(ch * zc, zc)])

    return count_kernel


def _combine_body(cnt_ref, invw_ref):
    r = invw_ref.shape[1]
    s = cnt_ref[0] + cnt_ref[1]
    invw_ref[...] = 1.0 / jnp.maximum(s[:, :r], 1.0)


def _combine(cnt16, r):
    nc, m, _ = cnt16.shape
    return pl.pallas_call(
        _combine_body,
        out_shape=jax.ShapeDtypeStruct((m, r), jnp.float32),
    )(cnt16)


def _make_edge_kernel(m_nodes, c_dim, r_rel, e_per_w, kc):
    zc = 200
    nchunk = m_nodes // zc
    iters = -(-nchunk // NS)
    nseg = m_nodes * r_rel
    mesh = plsc.VectorSubcoreMesh(core_axis_name="c", subcore_axis_name="s")

    @functools.partial(
        pl.kernel,
        out_type=jax.ShapeDtypeStruct((NC, m_nodes, c_dim), jnp.float32),
        mesh=mesh,
        scratch_types=[
            pltpu.VMEM((kc,), jnp.int32),
            pltpu.VMEM((kc,), jnp.int32),
            pltpu.VMEM((kc,), jnp.int32),
            pltpu.VMEM((kc,), jnp.int32),
            pltpu.VMEM((kc,), jnp.int32),
            pltpu.VMEM((kc,), jnp.float32),
            pltpu.VMEM((kc, c_dim), jnp.float32),
            pltpu.VMEM((200, c_dim), jnp.float32),
            pltpu.VMEM_SHARED((nseg,), jnp.float32),
            pltpu.VMEM_SHARED((m_nodes, c_dim), jnp.float32),
            pltpu.SemaphoreType.DMA,
        ],
        compiler_params=pltpu.CompilerParams(needs_layout_passes=False),
    )
    def edge_kernel(src_hbm, dst_hbm, typ_hbm, invw_hbm, z_hbm, out_hbm,
                    srcb, dstb, typb, zidxb, widxb, wb, zrows, bounce,
                    invw_sh, acc_sh, sem):
        cidx = lax.axis_index("c")
        sidx = lax.axis_index("s")
        wid = sidx * NC + cidx

        @pl.when(sidx == 0)
        def _():
            pltpu.sync_copy(invw_hbm, invw_sh)

        def zrow(i, carry):
            for j in range(c_dim // L):
                bounce[i, pl.ds(j * L, L)] = jnp.zeros((L,), jnp.float32)
            return carry

        lax.fori_loop(0, zc, zrow, 0)
        for t in range(iters):
            ch = sidx + t * NS

            @pl.when(ch < nchunk)
            def _():
                pltpu.sync_copy(bounce, acc_sh.at[pl.ds(ch * zc, zc)])
        plsc.subcore_barrier()

        base = wid * e_per_w

        def chunk(g, carry):
            off = base + g * kc
            pltpu.sync_copy(src_hbm.at[pl.ds(off, kc)], srcb)
            pltpu.sync_copy(dst_hbm.at[pl.ds(off, kc)], dstb)
            pltpu.sync_copy(typ_hbm.at[pl.ds(off, kc)], typb)
            for j in range(kc // L):
                sv = srcb[pl.ds(j * L, L)]
                dv = dstb[pl.ds(j * L, L)]
                tv = typb[pl.ds(j * L, L)]
                zidxb[pl.ds(j * L, L)] = sv * r_rel + tv
                widxb[pl.ds(j * L, L)] = dv * r_rel + tv
            pltpu.sync_copy(invw_sh.at[widxb], wb)
            pltpu.async_copy(z_hbm.at[zidxb], zrows, sem).wait()

            def scale(g2, carry2):
                wv = wb[pl.ds(g2 * L, L)]
                for e2 in range(L):
                    ws = wv[e2]
                    row = g2 * L + e2
                    for j in range(c_dim // L):
                        zrows[row, pl.ds(j * L, L)] = (
                            zrows[row, pl.ds(j * L, L)] * ws)
                return carry2

            lax.fori_loop(0, kc // L, scale, 0)
            pltpu.sync_copy(zrows, acc_sh.at[dstb], add=True)
            return carry

        lax.fori_loop(0, e_per_w // kc, chunk, 0)
        plsc.subcore_barrier()
        for t in range(iters):
            ch = sidx + t * NS

            @pl.when(ch < nchunk)
            def _():
                pltpu.sync_copy(acc_sh.at[pl.ds(ch * zc, zc)], bounce)
                pltpu.sync_copy(bounce, out_hbm.at[cidx, pl.ds(ch * zc, zc)])

    return edge_kernel


def _phase_c_body(outroot_ref, part_ref, g_ref, b_ref, hw_ref, hb_ref,
                  out_ref):
    ob = outroot_ref[...] + part_ref[0] + part_ref[1]
    m = jnp.mean(ob, axis=1, keepdims=True)
    v = jnp.mean((ob - m) * (ob - m), axis=1, keepdims=True)
    h2 = (ob - m) * lax.rsqrt(v + 1e-5) * g_ref[...] + b_ref[...]
    h2 = jnp.maximum(h2, 0.0)
    pooled = jnp.mean(h2, axis=0, keepdims=True)
    i = pl.program_id(0)
    out_ref[pl.ds(i, 1), :] = (
        jnp.dot(pooled, hw_ref[...], preferred_element_type=jnp.float32,
                precision=lax.Precision.HIGHEST)
        + hb_ref[...])


def _phase_c(outroot, partials, g, b, head_w_pad, head_b_pad, bsz, n):
    c = outroot.shape[1]
    cp = head_w_pad.shape[1]
    return pl.pallas_call(
        _phase_c_body,
        grid=(bsz,),
        in_specs=[
            pl.BlockSpec((n, c), lambda i: (i, 0)),
            pl.BlockSpec((NC, n, c), lambda i: (0, i, 0)),
            pl.BlockSpec((1, c), lambda i: (0, 0)),
            pl.BlockSpec((1, c), lambda i: (0, 0)),
            pl.BlockSpec((c, cp), lambda i: (0, 0)),
            pl.BlockSpec((1, cp), lambda i: (0, 0)),
        ],
        out_specs=pl.BlockSpec((bsz, cp), lambda i: (0, 0)),
        out_shape=jax.ShapeDtypeStruct((bsz, cp), jnp.float32),
    )(outroot, partials, g.reshape(1, c), b.reshape(1, c), head_w_pad,
      head_b_pad)


def kernel(x, batch_edge_index, batch_edge_types, ln1_g, ln1_b, W, root,
           bias, ln2_g, ln2_b, head_w, head_b):
    bsz, n, c = x.shape
    r = W.shape[0]
    s_out = head_w.shape[1]
    e = batch_edge_types.shape[0]
    m = bsz * n
    e_per_w = e // NW
    kc = 80

    xf2 = x.reshape(m, c)
    z, outroot = _phase_a(xf2, ln1_g, ln1_b, W, root, bias)

    src = batch_edge_index[0]
    dst = batch_edge_index[1]
    typ = batch_edge_types

    cnt16 = _make_count_kernel(m, e_per_w, kc)(dst, typ)
    invw = _combine(cnt16, r).reshape(m * r)
    partials = _make_edge_kernel(m, c, r, e_per_w, kc)(
        src, dst, typ, invw, z.reshape(m * r, c))

    cp = 128
    head_w_pad = jnp.pad(head_w, ((0, 0), (0, cp - s_out)))
    head_b_pad = jnp.pad(head_b, (0, cp - s_out)).reshape(1, cp)
    outpad = _phase_c(outroot, partials, ln2_g, ln2_b, head_w_pad,
                      head_b_pad, bsz, n)
    return outpad[:, :s_out]


# speedup vs baseline: 23.4771x; 1.6943x over previous
import functools

import jax
import jax.numpy as jnp
from jax import lax
from jax.experimental import pallas as pl
from jax.experimental.pallas import tpu as pltpu
from jax.experimental.pallas import tpu_sc as plsc

NC = 2
NS = 16
L = 16
NW = NC * NS


def _phase_a_body(x_ref, g_ref, b_ref, w_ref, root_ref, bias_ref,
                  z_ref, outroot_ref):
    xb = x_ref[...]
    m = jnp.mean(xb, axis=1, keepdims=True)
    v = jnp.mean((xb - m) * (xb - m), axis=1, keepdims=True)
    xf = (xb - m) * lax.rsqrt(v + 1e-5) * g_ref[...] + b_ref[...]
    outroot_ref[...] = (
        jnp.dot(xf, root_ref[...], preferred_element_type=jnp.float32,
                precision=lax.Precision.HIGHEST)
        + bias_ref[...])
    r = w_ref.shape[0]
    c = xb.shape[1]
    for i in range(r):
        z_ref[:, i * c:(i + 1) * c] = jnp.dot(
            xf, w_ref[i], preferred_element_type=jnp.float32,
            precision=lax.Precision.HIGHEST)


def _phase_a(xf2, g, b, w, root, bias):
    m, c = xf2.shape
    r = w.shape[0]
    bm = 1000
    grid = m // bm
    return pl.pallas_call(
        _phase_a_body,
        grid=(grid,),
        in_specs=[
            pl.BlockSpec((bm, c), lambda i: (i, 0)),
            pl.BlockSpec((1, c), lambda i: (0, 0)),
            pl.BlockSpec((1, c), lambda i: (0, 0)),
            pl.BlockSpec((r, c, c), lambda i: (0, 0, 0)),
            pl.BlockSpec((c, c), lambda i: (0, 0)),
            pl.BlockSpec((1, c), lambda i: (0, 0)),
        ],
        out_specs=[
            pl.BlockSpec((bm, r * c), lambda i: (i, 0)),
            pl.BlockSpec((bm, c), lambda i: (i, 0)),
        ],
        out_shape=[
            jax.ShapeDtypeStruct((m, r * c), jnp.float32),
            jax.ShapeDtypeStruct((m, c), jnp.float32),
        ],
    )(xf2, g.reshape(1, c), b.reshape(1, c), w, root, bias.reshape(1, c))


def _make_count_kernel(m_nodes, e_per_w, kc):
    zc = 200
    nchunk = m_nodes // zc
    iters = -(-nchunk // NS)
    mesh = plsc.VectorSubcoreMesh(core_axis_name="c", subcore_axis_name="s")

    @functools.partial(
        pl.kernel,
        out_type=jax.ShapeDtypeStruct((NC, m_nodes, 128), jnp.float32),
        mesh=mesh,
        scratch_types=[
            pltpu.VMEM((kc,), jnp.int32),
            pltpu.VMEM((kc,), jnp.int32),
            pltpu.VMEM((kc, 128), jnp.float32),
            pltpu.VMEM((zc, 128), jnp.float32),
            pltpu.VMEM_SHARED((m_nodes, 128), jnp.float32),
        ],
        compiler_params=pltpu.CompilerParams(needs_layout_passes=False),
    )
    def count_kernel(dst_hbm, typ_hbm, out_hbm, dstb, typb, stag, bounce,
                     cnt_sh):
        cidx = lax.axis_index("c")
        sidx = lax.axis_index("s")
        wid = sidx * NC + cidx

        for i in range(kc):
            for j in range(1, 8):
                stag[i, pl.ds(j * L, L)] = jnp.zeros((L,), jnp.float32)

        def zrow(i, carry):
            for j in range(8):
                bounce[i, pl.ds(j * L, L)] = jnp.zeros((L,), jnp.float32)
            return carry

        lax.fori_loop(0, zc, zrow, 0)
        for t in range(iters):
            ch = sidx + t * NS

            @pl.when(ch < nchunk)
            def _():
                pltpu.sync_copy(bounce, cnt_sh.at[pl.ds(ch * zc, zc)])
        plsc.subcore_barrier()

        base = wid * e_per_w

        def chunk(g, carry):
            off = base + g * kc
            pltpu.sync_copy(dst_hbm.at[pl.ds(off, kc)], dstb)
            pltpu.sync_copy(typ_hbm.at[pl.ds(off, kc)], typb)

            lanes = lax.iota(jnp.int32, L)
            for j in range(kc // L):
                tv = typb[pl.ds(j * L, L)]
                for e2 in range(L):
                    stag[j * L + e2, pl.ds(0, L)] = jnp.where(
                        lanes == tv[e2], 1.0, 0.0).astype(jnp.float32)
            pltpu.sync_copy(stag, cnt_sh.at[dstb], add=True)
            return carry

        lax.fori_loop(0, e_per_w // kc, chunk, 0)
        plsc.subcore_barrier()
        for t in range(iters):
            ch = sidx + t * NS

            @pl.when(ch < nchunk)
            def _():
                pltpu.sync_copy(cnt_sh.at[pl.ds(ch * zc, zc)], bounce)
                pltpu.sync_copy(bounce,
                                out_hbm.at[cidx, pl.ds(ch * zc, zc)])

    return count_kernel


def _combine_body(cnt_ref, invw_ref):
    r = invw_ref.shape[1]
    s = cnt_ref[0] + cnt_ref[1]
    invw_ref[...] = 1.0 / jnp.maximum(s[:, :r], 1.0)


def _combine(cnt16, r):
    nc, m, _ = cnt16.shape
    return pl.pallas_call(
        _combine_body,
        out_shape=jax.ShapeDtypeStruct((m, r), jnp.float32),
    )(cnt16)


def _make_idx_kernel(r_rel):
    def body(src_ref, dst_ref, typ_ref, zidx_ref, widx_ref):
        t = typ_ref[...]
        zidx_ref[...] = src_ref[...] * r_rel + t
        widx_ref[...] = dst_ref[...] * r_rel + t

    def run(src, dst, typ):
        e = src.shape[0]
        sh = (e // 128, 128)
        z, w = pl.pallas_call(
            body,
            out_shape=[jax.ShapeDtypeStruct(sh, jnp.int32),
                       jax.ShapeDtypeStruct(sh, jnp.int32)],
        )(src.reshape(sh), dst.reshape(sh), typ.reshape(sh))
        return z.reshape(e), w.reshape(e)

    return run


def _make_edge_kernel(m_nodes, c_dim, r_rel, e_per_w, kc):
    nchunk = m_nodes // kc
    iters = -(-nchunk // NS)
    nseg = m_nodes * r_rel
    ng = e_per_w // kc
    mesh = plsc.VectorSubcoreMesh(core_axis_name="c", subcore_axis_name="s")

    @functools.partial(
        pl.kernel,
        out_type=jax.ShapeDtypeStruct((NC, m_nodes, c_dim), jnp.float32),
        mesh=mesh,
        scratch_types=[
            pltpu.VMEM((e_per_w,), jnp.int32),
            pltpu.VMEM((e_per_w,), jnp.int32),
            pltpu.VMEM((kc,), jnp.int32),
            pltpu.VMEM((kc,), jnp.int32),
            pltpu.VMEM((kc,), jnp.int32),
            pltpu.VMEM((kc,), jnp.int32),
            pltpu.VMEM((kc,), jnp.int32),
            pltpu.VMEM((kc,), jnp.int32),
            pltpu.VMEM((kc,), jnp.float32),
            pltpu.VMEM((kc,), jnp.float32),
            pltpu.VMEM((kc, c_dim), jnp.float32),
            pltpu.VMEM((kc, c_dim), jnp.float32),
            pltpu.VMEM_SHARED((nseg,), jnp.float32),
            pltpu.VMEM_SHARED((m_nodes, c_dim), jnp.float32),
            pltpu.SemaphoreType.DMA,
            pltpu.SemaphoreType.DMA,
        ],
        compiler_params=pltpu.CompilerParams(needs_layout_passes=False),
    )
    def edge_kernel(zidx_hbm, widx_hbm, invw_hbm, z_hbm, out_hbm,
                    zixall, wixall, zb0, zb1, wib0, wib1, db0, db1,
                    wb0, wb1, zr0, zr1, invw_sh, acc_sh, sem0, sem1):
        cidx = lax.axis_index("c")
        sidx = lax.axis_index("s")
        wid = sidx * NC + cidx
        base = wid * e_per_w

        @pl.when(sidx == 0)
        def _():
            pltpu.sync_copy(invw_hbm, invw_sh)
        pltpu.sync_copy(zidx_hbm.at[pl.ds(base, e_per_w)], zixall)
        pltpu.sync_copy(widx_hbm.at[pl.ds(base, e_per_w)], wixall)

        def zrow(i, carry):
            for j in range(c_dim // L):
                zr0[i, pl.ds(j * L, L)] = jnp.zeros((L,), jnp.float32)
            return carry

        lax.fori_loop(0, kc, zrow, 0)
        for t in range(iters):
            ch = sidx + t * NS

            @pl.when(ch < nchunk)
            def _():
                pltpu.sync_copy(zr0, acc_sh.at[pl.ds(ch * kc, kc)])
        plsc.subcore_barrier()

        bufs = ((zb0, wib0, db0, wb0, zr0, sem0),
                (zb1, wib1, db1, wb1, zr1, sem1))

        def prepare(g, zb, wib, db, wb, zr, sem):
            qb = g * kc
            for j in range(kc // L):
                zv = zixall[pl.ds(qb + j * L, L)]
                wv = wixall[pl.ds(qb + j * L, L)]
                zb[pl.ds(j * L, L)] = zv
                wib[pl.ds(j * L, L)] = wv
                db[pl.ds(j * L, L)] = lax.shift_right_logical(wv, 3)
            pltpu.sync_copy(invw_sh.at[wib], wb)
            pltpu.async_copy(z_hbm.at[zb], zr, sem)

        def drain(zb, wib, db, wb, zr, sem):
            pltpu.make_async_copy(z_hbm.at[zb], zr, sem).wait()

            def scale(g2, carry2):
                wv = wb[pl.ds(g2 * L, L)]
                for e2 in range(L):
                    ws = wv[e2]
                    row = g2 * L + e2
                    for j in range(c_dim // L):
                        zr[row, pl.ds(j * L, L)] = (
                            zr[row, pl.ds(j * L, L)] * ws)
                return carry2

            lax.fori_loop(0, kc // L, scale, 0)
            pltpu.sync_copy(zr, acc_sh.at[db], add=True)

        def step(g, carry):
            par = lax.rem(g, 2)

            @pl.when((g < ng) & (par == 0))
            def _():
                prepare(g, *bufs[0])

            @pl.when((g < ng) & (par == 1))
            def _():
                prepare(g, *bufs[1])

            @pl.when((g > 0) & (par == 1))
            def _():
                drain(*bufs[0])

            @pl.when((g > 0) & (par == 0))
            def _():
                drain(*bufs[1])

            return carry

        lax.fori_loop(0, ng + 1, step, 0)
        plsc.subcore_barrier()
        for t in range(iters):
            ch = sidx + t * NS

            @pl.when(ch < nchunk)
            def _():
                pltpu.sync_copy(acc_sh.at[pl.ds(ch * kc, kc)], zr0)
                pltpu.sync_copy(zr0, out_hbm.at[cidx, pl.ds(ch * kc, kc)])

    return edge_kernel


def _phase_c_body(outroot_ref, part_ref, g_ref, b_ref, hw_ref, hb_ref,
                  out_ref):
    ob = outroot_ref[...] + part_ref[0] + part_ref[1]
    m = jnp.mean(ob, axis=1, keepdims=True)
    v = jnp.mean((ob - m) * (ob - m), axis=1, keepdims=True)
    h2 = (ob - m) * lax.rsqrt(v + 1e-5) * g_ref[...] + b_ref[...]
    h2 = jnp.maximum(h2, 0.0)
    pooled = jnp.mean(h2, axis=0, keepdims=True)
    i = pl.program_id(0)
    out_ref[pl.ds(i, 1), :] = (
        jnp.dot(pooled, hw_ref[...], preferred_element_type=jnp.float32,
                precision=lax.Precision.HIGHEST)
        + hb_ref[...])


def _phase_c(outroot, partials, g, b, head_w_pad, head_b_pad, bsz, n):
    c = outroot.shape[1]
    cp = head_w_pad.shape[1]
    return pl.pallas_call(
        _phase_c_body,
        grid=(bsz,),
        in_specs=[
            pl.BlockSpec((n, c), lambda i: (i, 0)),
            pl.BlockSpec((NC, n, c), lambda i: (0, i, 0)),
            pl.BlockSpec((1, c), lambda i: (0, 0)),
            pl.BlockSpec((1, c), lambda i: (0, 0)),
            pl.BlockSpec((c, cp), lambda i: (0, 0)),
            pl.BlockSpec((1, cp), lambda i: (0, 0)),
        ],
        out_specs=pl.BlockSpec((bsz, cp), lambda i: (0, 0)),
        out_shape=jax.ShapeDtypeStruct((bsz, cp), jnp.float32),
    )(outroot, partials, g.reshape(1, c), b.reshape(1, c), head_w_pad,
      head_b_pad)


def kernel(x, batch_edge_index, batch_edge_types, ln1_g, ln1_b, W, root,
           bias, ln2_g, ln2_b, head_w, head_b):
    bsz, n, c = x.shape
    r = W.shape[0]
    s_out = head_w.shape[1]
    e = batch_edge_types.shape[0]
    m = bsz * n
    e_per_w = e // NW
    kc = 80

    xf2 = x.reshape(m, c)
    z, outroot = _phase_a(xf2, ln1_g, ln1_b, W, root, bias)

    src = batch_edge_index[0]
    dst = batch_edge_index[1]
    typ = batch_edge_types

    cnt16 = _make_count_kernel(m, e_per_w, kc)(dst, typ)
    invw = _combine(cnt16, r).reshape(m * r)
    zidx, widx = _make_idx_kernel(r)(src, dst, typ)
    partials = _make_edge_kernel(m, c, r, e_per_w, kc)(
        zidx, widx, invw, z.reshape(m * r, c))

    cp = 128
    head_w_pad = jnp.pad(head_w, ((0, 0), (0, cp - s_out)))
    head_b_pad = jnp.pad(head_b, (0, cp - s_out)).reshape(1, cp)
    outpad = _phase_c(outroot, partials, ln2_g, ln2_b, head_w_pad,
                      head_b_pad, bsz, n)
    return outpad[:, :s_out]


# speedup vs baseline: 29.4500x; 1.2544x over previous
import functools

import jax
import jax.numpy as jnp
from jax import lax
from jax.experimental import pallas as pl
from jax.experimental.pallas import tpu as pltpu
from jax.experimental.pallas import tpu_sc as plsc

NC = 2
NS = 16
L = 16
NW = NC * NS


def _phase_a_body(x_ref, g_ref, b_ref, w_ref, root_ref, bias_ref,
                  z_ref, outroot_ref):
    xb = x_ref[...]
    m = jnp.mean(xb, axis=1, keepdims=True)
    v = jnp.mean((xb - m) * (xb - m), axis=1, keepdims=True)
    xf = (xb - m) * lax.rsqrt(v + 1e-5) * g_ref[...] + b_ref[...]
    outroot_ref[...] = (
        jnp.dot(xf, root_ref[...], preferred_element_type=jnp.float32,
                precision=lax.Precision.HIGHEST)
        + bias_ref[...])
    r = w_ref.shape[0]
    c = xb.shape[1]
    for i in range(r):
        z_ref[:, i * c:(i + 1) * c] = jnp.dot(
            xf, w_ref[i], preferred_element_type=jnp.float32,
            precision=lax.Precision.HIGHEST)


def _phase_a(xf2, g, b, w, root, bias):
    m, c = xf2.shape
    r = w.shape[0]
    bm = 1000
    grid = m // bm
    return pl.pallas_call(
        _phase_a_body,
        grid=(grid,),
        in_specs=[
            pl.BlockSpec((bm, c), lambda i: (i, 0)),
            pl.BlockSpec((1, c), lambda i: (0, 0)),
            pl.BlockSpec((1, c), lambda i: (0, 0)),
            pl.BlockSpec((r, c, c), lambda i: (0, 0, 0)),
            pl.BlockSpec((c, c), lambda i: (0, 0)),
            pl.BlockSpec((1, c), lambda i: (0, 0)),
        ],
        out_specs=[
            pl.BlockSpec((bm, r * c), lambda i: (i, 0)),
            pl.BlockSpec((bm, c), lambda i: (i, 0)),
        ],
        out_shape=[
            jax.ShapeDtypeStruct((m, r * c), jnp.float32),
            jax.ShapeDtypeStruct((m, c), jnp.float32),
        ],
    )(xf2, g.reshape(1, c), b.reshape(1, c), w, root, bias.reshape(1, c))


def _make_count_kernel(m_nodes, r_rel, e_per_w):
    nrow = 640
    nseg_pad = nrow * 128
    rpt = nrow // NS
    mesh = plsc.VectorSubcoreMesh(core_axis_name="c", subcore_axis_name="s")

    @functools.partial(
        pl.kernel,
        out_type=jax.ShapeDtypeStruct((NC, nrow, 128), jnp.float32),
        mesh=mesh,
        scratch_types=[
            pltpu.VMEM((e_per_w,), jnp.int32),
            pltpu.VMEM((nrow, 128), jnp.float32),
            pltpu.VMEM((rpt, 128), jnp.float32),
            [pltpu.VMEM((128,), jnp.int32) for _ in range(5)],
            pltpu.VMEM_SHARED((nrow, 128), jnp.float32),
        ],
        compiler_params=pltpu.CompilerParams(needs_layout_passes=False),
    )
    def count_kernel(widx_hbm, out_hbm, wixall, cnt, stage, idrows, cnt_sh):
        cidx = lax.axis_index("c")
        sidx = lax.axis_index("s")
        wid = sidx * NC + cidx
        base = wid * e_per_w

        pltpu.sync_copy(widx_hbm.at[pl.ds(base, e_per_w)], wixall)

        for k in range(5):
            for g in range(8):
                idrows[k][pl.ds(g * L, L)] = (
                    lax.iota(jnp.int32, L) + k * 128 + g * L)

        def zrow(i, carry):
            for j in range(8):
                cnt[i, pl.ds(j * L, L)] = jnp.zeros((L,), jnp.float32)
            return carry

        lax.fori_loop(0, nrow, zrow, 0)
        pltpu.sync_copy(cnt.at[pl.ds(0, rpt)],
                        cnt_sh.at[pl.ds(sidx * rpt, rpt)])
        plsc.subcore_barrier()

        def acc(g, carry):
            wv = wixall[pl.ds(g * L, L)]
            row = lax.shift_right_logical(wv, 7)
            col = lax.bitwise_and(wv, 127)
            plsc.addupdate_scatter(cnt, [row, col],
                                   jnp.ones((L,), jnp.float32))
            return carry

        lax.fori_loop(0, e_per_w // L, acc, 0)

        for k in range(5):
            pltpu.sync_copy(cnt.at[pl.ds(k * 128, 128)],
                            cnt_sh.at[idrows[k]], add=True)
        plsc.subcore_barrier()

        pltpu.sync_copy(cnt_sh.at[pl.ds(sidx * rpt, rpt)], stage)
        pltpu.sync_copy(stage, out_hbm.at[cidx, pl.ds(sidx * rpt, rpt)])

    return count_kernel


def _combine_body(cnt_ref, invw_ref):
    s = cnt_ref[0] + cnt_ref[1]
    invw_ref[...] = 1.0 / jnp.maximum(s, 1.0)


def _combine(cnt2):
    nc, nrow, ncol = cnt2.shape
    return pl.pallas_call(
        _combine_body,
        out_shape=jax.ShapeDtypeStruct((nrow, ncol), jnp.float32),
    )(cnt2)


def _make_idx_kernel(r_rel):
    def body(src_ref, dst_ref, typ_ref, zidx_ref, widx_ref):
        t = typ_ref[...]
        zidx_ref[...] = src_ref[...] * r_rel + t
        widx_ref[...] = dst_ref[...] * r_rel + t

    def run(src, dst, typ):
        e = src.shape[0]
        sh = (e // 128, 128)
        z, w = pl.pallas_call(
            body,
            out_shape=[jax.ShapeDtypeStruct(sh, jnp.int32),
                       jax.ShapeDtypeStruct(sh, jnp.int32)],
        )(src.reshape(sh), dst.reshape(sh), typ.reshape(sh))
        return z.reshape(e), w.reshape(e)

    return run


def _make_edge_kernel(m_nodes, c_dim, r_rel, e_per_w, kc, nseg):
    nchunk = m_nodes // kc
    iters = -(-nchunk // NS)
    ng = e_per_w // kc
    mesh = plsc.VectorSubcoreMesh(core_axis_name="c", subcore_axis_name="s")

    @functools.partial(
        pl.kernel,
        out_type=jax.ShapeDtypeStruct((NC, m_nodes, c_dim), jnp.float32),
        mesh=mesh,
        scratch_types=[
            pltpu.VMEM((e_per_w,), jnp.int32),
            pltpu.VMEM((e_per_w,), jnp.int32),
            pltpu.VMEM((kc,), jnp.int32),
            pltpu.VMEM((kc,), jnp.int32),
            pltpu.VMEM((kc,), jnp.int32),
            pltpu.VMEM((kc,), jnp.int32),
            pltpu.VMEM((kc,), jnp.int32),
            pltpu.VMEM((kc,), jnp.int32),
            pltpu.VMEM((kc,), jnp.float32),
            pltpu.VMEM((kc,), jnp.float32),
            pltpu.VMEM((kc, c_dim), jnp.float32),
            pltpu.VMEM((kc, c_dim), jnp.float32),
            pltpu.VMEM_SHARED((nseg,), jnp.float32),
            pltpu.VMEM_SHARED((m_nodes, c_dim), jnp.float32),
            pltpu.SemaphoreType.DMA,
            pltpu.SemaphoreType.DMA,
        ],
        compiler_params=pltpu.CompilerParams(needs_layout_passes=False),
    )
    def edge_kernel(zidx_hbm, widx_hbm, invw_hbm, z_hbm, out_hbm,
                    zixall, wixall, zb0, zb1, wib0, wib1, db0, db1,
                    wb0, wb1, zr0, zr1, invw_sh, acc_sh, sem0, sem1):
        cidx = lax.axis_index("c")
        sidx = lax.axis_index("s")
        wid = sidx * NC + cidx
        base = wid * e_per_w

        @pl.when(sidx == 0)
        def _():
            pltpu.sync_copy(invw_hbm, invw_sh)
        pltpu.sync_copy(zidx_hbm.at[pl.ds(base, e_per_w)], zixall)
        pltpu.sync_copy(widx_hbm.at[pl.ds(base, e_per_w)], wixall)

        def zrow(i, carry):
            for j in range(c_dim // L):
                zr0[i, pl.ds(j * L, L)] = jnp.zeros((L,), jnp.float32)
            return carry

        lax.fori_loop(0, kc, zrow, 0)
        for t in range(iters):
            ch = sidx + t * NS

            @pl.when(ch < nchunk)
            def _():
                pltpu.sync_copy(zr0, acc_sh.at[pl.ds(ch * kc, kc)])
        plsc.subcore_barrier()

        bufs = ((zb0, wib0, db0, wb0, zr0, sem0),
                (zb1, wib1, db1, wb1, zr1, sem1))

        def prepare(g, zb, wib, db, wb, zr, sem):
            qb = g * kc
            for j in range(kc // L):
                zv = zixall[pl.ds(qb + j * L, L)]
                wv = wixall[pl.ds(qb + j * L, L)]
                zb[pl.ds(j * L, L)] = zv
                wib[pl.ds(j * L, L)] = wv
                db[pl.ds(j * L, L)] = lax.shift_right_logical(wv, 3)
            pltpu.sync_copy(invw_sh.at[wib], wb)
            pltpu.async_copy(z_hbm.at[zb], zr, sem)

        def drain(zb, wib, db, wb, zr, sem):
            pltpu.make_async_copy(z_hbm.at[zb], zr, sem).wait()

            def scale(g2, carry2):
                wv = wb[pl.ds(g2 * L, L)]
                for e2 in range(L):
                    ws = wv[e2]
                    row = g2 * L + e2
                    for j in range(c_dim // L):
                        zr[row, pl.ds(j * L, L)] = (
                            zr[row, pl.ds(j * L, L)] * ws)
                return carry2

            lax.fori_loop(0, kc // L, scale, 0)
            pltpu.sync_copy(zr, acc_sh.at[db], add=True)

        def step(g, carry):
            par = lax.rem(g, 2)

            @pl.when((g < ng) & (par == 0))
            def _():
                prepare(g, *bufs[0])

            @pl.when((g < ng) & (par == 1))
            def _():
                prepare(g, *bufs[1])

            @pl.when((g > 0) & (par == 1))
            def _():
                drain(*bufs[0])

            @pl.when((g > 0) & (par == 0))
            def _():
                drain(*bufs[1])

            return carry

        lax.fori_loop(0, ng + 1, step, 0)
        plsc.subcore_barrier()
        for t in range(iters):
            ch = sidx + t * NS

            @pl.when(ch < nchunk)
            def _():
                pltpu.sync_copy(acc_sh.at[pl.ds(ch * kc, kc)], zr0)
                pltpu.sync_copy(zr0, out_hbm.at[cidx, pl.ds(ch * kc, kc)])

    return edge_kernel


def _phase_c_body(outroot_ref, part_ref, g_ref, b_ref, hw_ref, hb_ref,
                  out_ref):
    ob = outroot_ref[...] + part_ref[0] + part_ref[1]
    m = jnp.mean(ob, axis=1, keepdims=True)
    v = jnp.mean((ob - m) * (ob - m), axis=1, keepdims=True)
    h2 = (ob - m) * lax.rsqrt(v + 1e-5) * g_ref[...] + b_ref[...]
    h2 = jnp.maximum(h2, 0.0)
    pooled = jnp.mean(h2, axis=0, keepdims=True)
    i = pl.program_id(0)
    out_ref[pl.ds(i, 1), :] = (
        jnp.dot(pooled, hw_ref[...], preferred_element_type=jnp.float32,
                precision=lax.Precision.HIGHEST)
        + hb_ref[...])


def _phase_c(outroot, partials, g, b, head_w_pad, head_b_pad, bsz, n):
    c = outroot.shape[1]
    cp = head_w_pad.shape[1]
    return pl.pallas_call(
        _phase_c_body,
        grid=(bsz,),
        in_specs=[
            pl.BlockSpec((n, c), lambda i: (i, 0)),
            pl.BlockSpec((NC, n, c), lambda i: (0, i, 0)),
            pl.BlockSpec((1, c), lambda i: (0, 0)),
            pl.BlockSpec((1, c), lambda i: (0, 0)),
            pl.BlockSpec((c, cp), lambda i: (0, 0)),
            pl.BlockSpec((1, cp), lambda i: (0, 0)),
        ],
        out_specs=pl.BlockSpec((bsz, cp), lambda i: (0, 0)),
        out_shape=jax.ShapeDtypeStruct((bsz, cp), jnp.float32),
    )(outroot, partials, g.reshape(1, c), b.reshape(1, c), head_w_pad,
      head_b_pad)


def kernel(x, batch_edge_index, batch_edge_types, ln1_g, ln1_b, W, root,
           bias, ln2_g, ln2_b, head_w, head_b):
    bsz, n, c = x.shape
    r = W.shape[0]
    s_out = head_w.shape[1]
    e = batch_edge_types.shape[0]
    m = bsz * n
    e_per_w = e // NW
    kc = 80

    xf2 = x.reshape(m, c)
    z, outroot = _phase_a(xf2, ln1_g, ln1_b, W, root, bias)

    src = batch_edge_index[0]
    dst = batch_edge_index[1]
    typ = batch_edge_types

    zidx, widx = _make_idx_kernel(r)(src, dst, typ)
    cnt2 = _make_count_kernel(m, r, e_per_w)(widx)
    invw = _combine(cnt2).reshape(-1)
    partials = _make_edge_kernel(m, c, r, e_per_w, kc, invw.shape[0])(
        zidx, widx, invw, z.reshape(m * r, c))

    cp = 128
    head_w_pad = jnp.pad(head_w, ((0, 0), (0, cp - s_out)))
    head_b_pad = jnp.pad(head_b, (0, cp - s_out)).reshape(1, cp)
    outpad = _phase_c(outroot, partials, ln2_g, ln2_b, head_w_pad,
                      head_b_pad, bsz, n)
    return outpad[:, :s_out]


# speedup vs baseline: 29.8345x; 1.0131x over previous
import functools

import jax
import jax.numpy as jnp
from jax import lax
from jax.experimental import pallas as pl
from jax.experimental.pallas import tpu as pltpu
from jax.experimental.pallas import tpu_sc as plsc

NC = 2
NS = 16
L = 16
NW = NC * NS


def _phase_a_body(x_ref, g_ref, b_ref, w_ref, root_ref, bias_ref,
                  src_ref, dst_ref, typ_ref,
                  z_ref, outroot_ref, zidx_ref, widx_ref):
    @pl.when(pl.program_id(0) == 0)
    def _():
        t = typ_ref[...]
        rr = w_ref.shape[0]
        zidx_ref[...] = src_ref[...] * rr + t
        widx_ref[...] = dst_ref[...] * rr + t

    xb = x_ref[...]
    m = jnp.mean(xb, axis=1, keepdims=True)
    v = jnp.mean((xb - m) * (xb - m), axis=1, keepdims=True)
    xf = (xb - m) * lax.rsqrt(v + 1e-5) * g_ref[...] + b_ref[...]
    outroot_ref[...] = (
        jnp.dot(xf, root_ref[...], preferred_element_type=jnp.float32,
                precision=lax.Precision.HIGHEST)
        + bias_ref[...])
    r = w_ref.shape[0]
    c = xb.shape[1]
    for i in range(r):
        z_ref[:, i * c:(i + 1) * c] = jnp.dot(
            xf, w_ref[i], preferred_element_type=jnp.float32,
            precision=lax.Precision.HIGHEST)


def _phase_a(xf2, g, b, w, root, bias, src, dst, typ):
    m, c = xf2.shape
    r = w.shape[0]
    e = src.shape[0]
    bm = 1000
    grid = m // bm
    er = e // 128
    eb = er // grid
    esh = (er, 128)
    z, outroot, zidx, widx = pl.pallas_call(
        _phase_a_body,
        grid=(grid,),
        in_specs=[
            pl.BlockSpec((bm, c), lambda i: (i, 0)),
            pl.BlockSpec((1, c), lambda i: (0, 0)),
            pl.BlockSpec((1, c), lambda i: (0, 0)),
            pl.BlockSpec((r, c, c), lambda i: (0, 0, 0)),
            pl.BlockSpec((c, c), lambda i: (0, 0)),
            pl.BlockSpec((1, c), lambda i: (0, 0)),
            pl.BlockSpec((er, 128), lambda i: (0, 0)),
            pl.BlockSpec((er, 128), lambda i: (0, 0)),
            pl.BlockSpec((er, 128), lambda i: (0, 0)),
        ],
        out_specs=[
            pl.BlockSpec((bm, r * c), lambda i: (i, 0)),
            pl.BlockSpec((bm, c), lambda i: (i, 0)),
            pl.BlockSpec((er, 128), lambda i: (0, 0)),
            pl.BlockSpec((er, 128), lambda i: (0, 0)),
        ],
        out_shape=[
            jax.ShapeDtypeStruct((m, r * c), jnp.float32),
            jax.ShapeDtypeStruct((m, c), jnp.float32),
            jax.ShapeDtypeStruct(esh, jnp.int32),
            jax.ShapeDtypeStruct(esh, jnp.int32),
        ],
    )(xf2, g.reshape(1, c), b.reshape(1, c), w, root, bias.reshape(1, c),
      src.reshape(esh), dst.reshape(esh), typ.reshape(esh))
    return z, outroot, zidx.reshape(e), widx.reshape(e)


def _make_count_kernel(m_nodes, r_rel, e_per_w):
    nrow = 640
    nseg_pad = nrow * 128
    rpt = nrow // NS
    mesh = plsc.VectorSubcoreMesh(core_axis_name="c", subcore_axis_name="s")

    @functools.partial(
        pl.kernel,
        out_type=jax.ShapeDtypeStruct((NC, nrow, 128), jnp.float32),
        mesh=mesh,
        scratch_types=[
            pltpu.VMEM((e_per_w,), jnp.int32),
            pltpu.VMEM((nrow, 128), jnp.float32),
            pltpu.VMEM((rpt, 128), jnp.float32),
            [pltpu.VMEM((128,), jnp.int32) for _ in range(5)],
            pltpu.VMEM_SHARED((nrow, 128), jnp.float32),
        ],
        compiler_params=pltpu.CompilerParams(needs_layout_passes=False),
    )
    def count_kernel(widx_hbm, out_hbm, wixall, cnt, stage, idrows, cnt_sh):
        cidx = lax.axis_index("c")
        sidx = lax.axis_index("s")
        wid = sidx * NC + cidx
        base = wid * e_per_w

        pltpu.sync_copy(widx_hbm.at[pl.ds(base, e_per_w)], wixall)

        for k in range(5):
            for g in range(8):
                idrows[k][pl.ds(g * L, L)] = (
                    lax.iota(jnp.int32, L) + k * 128 + g * L)

        def zrow(i, carry):
            for j in range(8):
                cnt[i, pl.ds(j * L, L)] = jnp.zeros((L,), jnp.float32)
            return carry

        lax.fori_loop(0, nrow, zrow, 0)
        pltpu.sync_copy(cnt.at[pl.ds(0, rpt)],
                        cnt_sh.at[pl.ds(sidx * rpt, rpt)])
        plsc.subcore_barrier()

        def acc(g, carry):
            wv = wixall[pl.ds(g * L, L)]
            row = lax.shift_right_logical(wv, 7)
            col = lax.bitwise_and(wv, 127)
            plsc.addupdate_scatter(cnt, [row, col],
                                   jnp.ones((L,), jnp.float32))
            return carry

        lax.fori_loop(0, e_per_w // L, acc, 0)

        for k in range(5):
            pltpu.sync_copy(cnt.at[pl.ds(k * 128, 128)],
                            cnt_sh.at[idrows[k]], add=True)
        plsc.subcore_barrier()

        pltpu.sync_copy(cnt_sh.at[pl.ds(sidx * rpt, rpt)], stage)
        pltpu.sync_copy(stage, out_hbm.at[cidx, pl.ds(sidx * rpt, rpt)])

    return count_kernel


def _combine_body(cnt_ref, invw_ref):
    s = cnt_ref[0] + cnt_ref[1]
    invw_ref[...] = 1.0 / jnp.maximum(s, 1.0)


def _combine(cnt2):
    nc, nrow, ncol = cnt2.shape
    return pl.pallas_call(
        _combine_body,
        out_shape=jax.ShapeDtypeStruct((nrow, ncol), jnp.float32),
    )(cnt2)


def _make_edge_kernel(m_nodes, c_dim, r_rel, e_per_w, kc, nseg):
    nchunk = m_nodes // kc
    iters = -(-nchunk // NS)
    ng = e_per_w // kc
    mesh = plsc.VectorSubcoreMesh(core_axis_name="c", subcore_axis_name="s")

    @functools.partial(
        pl.kernel,
        out_type=jax.ShapeDtypeStruct((NC, m_nodes, c_dim), jnp.float32),
        mesh=mesh,
        scratch_types=[
            pltpu.VMEM((e_per_w,), jnp.int32),
            pltpu.VMEM((e_per_w,), jnp.int32),
            pltpu.VMEM((kc,), jnp.int32),
            pltpu.VMEM((kc,), jnp.int32),
            pltpu.VMEM((kc,), jnp.int32),
            pltpu.VMEM((kc,), jnp.int32),
            pltpu.VMEM((kc,), jnp.int32),
            pltpu.VMEM((kc,), jnp.int32),
            pltpu.VMEM((kc,), jnp.float32),
            pltpu.VMEM((kc,), jnp.float32),
            pltpu.VMEM((kc, c_dim), jnp.float32),
            pltpu.VMEM((kc, c_dim), jnp.float32),
            pltpu.VMEM_SHARED((nseg,), jnp.float32),
            pltpu.VMEM_SHARED((m_nodes, c_dim), jnp.float32),
            pltpu.SemaphoreType.DMA,
            pltpu.SemaphoreType.DMA,
            pltpu.SemaphoreType.DMA,
            pltpu.SemaphoreType.DMA,
        ],
        compiler_params=pltpu.CompilerParams(needs_layout_passes=False),
    )
    def edge_kernel(zidx_hbm, widx_hbm, invw_hbm, z_hbm, out_hbm,
                    zixall, wixall, zb0, zb1, wib0, wib1, db0, db1,
                    wb0, wb1, zr0, zr1, invw_sh, acc_sh, sem0, sem1,
                    ssem0, ssem1):
        cidx = lax.axis_index("c")
        sidx = lax.axis_index("s")
        wid = sidx * NC + cidx
        base = wid * e_per_w

        @pl.when(sidx == 0)
        def _():
            pltpu.sync_copy(invw_hbm, invw_sh)
        pltpu.sync_copy(zidx_hbm.at[pl.ds(base, e_per_w)], zixall)
        pltpu.sync_copy(widx_hbm.at[pl.ds(base, e_per_w)], wixall)

        def zrow(i, carry):
            for j in range(c_dim // L):
                zr0[i, pl.ds(j * L, L)] = jnp.zeros((L,), jnp.float32)
            return carry

        lax.fori_loop(0, kc, zrow, 0)
        for t in range(iters):
            ch = sidx + t * NS

            @pl.when(ch < nchunk)
            def _():
                pltpu.sync_copy(zr0, acc_sh.at[pl.ds(ch * kc, kc)])
        plsc.subcore_barrier()

        bufs = ((zb0, wib0, db0, wb0, zr0, sem0, ssem0),
                (zb1, wib1, db1, wb1, zr1, sem1, ssem1))

        def prepare(g, zb, wib, db, wb, zr, sem, ssem):
            @pl.when(g >= 2)
            def _():
                pltpu.make_async_copy(zr, acc_sh.at[db], ssem).wait()
            qb = g * kc
            for j in range(kc // L):
                zv = zixall[pl.ds(qb + j * L, L)]
                wv = wixall[pl.ds(qb + j * L, L)]
                zb[pl.ds(j * L, L)] = zv
                wib[pl.ds(j * L, L)] = wv
                db[pl.ds(j * L, L)] = lax.shift_right_logical(wv, 3)
            pltpu.sync_copy(invw_sh.at[wib], wb)
            pltpu.async_copy(z_hbm.at[zb], zr, sem)

        def drain(zb, wib, db, wb, zr, sem, ssem):
            pltpu.make_async_copy(z_hbm.at[zb], zr, sem).wait()

            def scale(g2, carry2):
                wv = wb[pl.ds(g2 * L, L)]
                for e2 in range(L):
                    ws = wv[e2]
                    row = g2 * L + e2
                    for j in range(c_dim // L):
                        zr[row, pl.ds(j * L, L)] = (
                            zr[row, pl.ds(j * L, L)] * ws)
                return carry2

            lax.fori_loop(0, kc // L, scale, 0)
            pltpu.async_copy(zr, acc_sh.at[db], ssem, add=True)

        def step(g, carry):
            par = lax.rem(g, 2)

            @pl.when((g < ng) & (par == 0))
            def _():
                prepare(g, *bufs[0])

            @pl.when((g < ng) & (par == 1))
            def _():
                prepare(g, *bufs[1])

            @pl.when((g > 0) & (par == 1))
            def _():
                drain(*bufs[0])

            @pl.when((g > 0) & (par == 0))
            def _():
                drain(*bufs[1])

            return carry

        lax.fori_loop(0, ng + 1, step, 0)
        pltpu.make_async_copy(zr0, acc_sh.at[db0], ssem0).wait()
        pltpu.make_async_copy(zr1, acc_sh.at[db1], ssem1).wait()
        plsc.subcore_barrier()
        for t in range(iters):
            ch = sidx + t * NS

            @pl.when(ch < nchunk)
            def _():
                pltpu.sync_copy(acc_sh.at[pl.ds(ch * kc, kc)], zr0)
                pltpu.sync_copy(zr0, out_hbm.at[cidx, pl.ds(ch * kc, kc)])

    return edge_kernel


def _phase_c_body(outroot_ref, part_ref, g_ref, b_ref, hw_ref, hb_ref,
                  out_ref):
    ob = outroot_ref[...] + part_ref[0] + part_ref[1]
    m = jnp.mean(ob, axis=1, keepdims=True)
    v = jnp.mean((ob - m) * (ob - m), axis=1, keepdims=True)
    h2 = (ob - m) * lax.rsqrt(v + 1e-5) * g_ref[...] + b_ref[...]
    h2 = jnp.maximum(h2, 0.0)
    pooled = jnp.mean(h2, axis=0, keepdims=True)
    i = pl.program_id(0)
    out_ref[pl.ds(i, 1), :] = (
        jnp.dot(pooled, hw_ref[...], preferred_element_type=jnp.float32,
                precision=lax.Precision.HIGHEST)
        + hb_ref[...])


def _phase_c(outroot, partials, g, b, head_w_pad, head_b_pad, bsz, n):
    c = outroot.shape[1]
    cp = head_w_pad.shape[1]
    return pl.pallas_call(
        _phase_c_body,
        grid=(bsz,),
        in_specs=[
            pl.BlockSpec((n, c), lambda i: (i, 0)),
            pl.BlockSpec((NC, n, c), lambda i: (0, i, 0)),
            pl.BlockSpec((1, c), lambda i: (0, 0)),
            pl.BlockSpec((1, c), lambda i: (0, 0)),
            pl.BlockSpec((c, cp), lambda i: (0, 0)),
            pl.BlockSpec((1, cp), lambda i: (0, 0)),
        ],
        out_specs=pl.BlockSpec((bsz, cp), lambda i: (0, 0)),
        out_shape=jax.ShapeDtypeStruct((bsz, cp), jnp.float32),
    )(outroot, partials, g.reshape(1, c), b.reshape(1, c), head_w_pad,
      head_b_pad)


def kernel(x, batch_edge_index, batch_edge_types, ln1_g, ln1_b, W, root,
           bias, ln2_g, ln2_b, head_w, head_b):
    bsz, n, c = x.shape
    r = W.shape[0]
    s_out = head_w.shape[1]
    e = batch_edge_types.shape[0]
    m = bsz * n
    e_per_w = e // NW
    kc = 80

    xf2 = x.reshape(m, c)
    src = batch_edge_index[0]
    dst = batch_edge_index[1]
    typ = batch_edge_types

    z, outroot, zidx, widx = _phase_a(xf2, ln1_g, ln1_b, W, root, bias,
                                      src, dst, typ)
    cnt2 = _make_count_kernel(m, r, e_per_w)(widx)
    invw = _combine(cnt2).reshape(-1)
    partials = _make_edge_kernel(m, c, r, e_per_w, kc, invw.shape[0])(
        zidx, widx, invw, z.reshape(m * r, c))

    cp = 128
    head_w_pad = jnp.pad(head_w, ((0, 0), (0, cp - s_out)))
    head_b_pad = jnp.pad(head_b, (0, cp - s_out)).reshape(1, cp)
    outpad = _phase_c(outroot, partials, ln2_g, ln2_b, head_w_pad,
                      head_b_pad, bsz, n)
    return outpad[:, :s_out]
